# baseline (speedup 1.0000x reference)
import jax
import jax.numpy as jnp
from jax import lax
from jax.experimental import pallas as pl
from jax.experimental.pallas import tpu as pltpu
from jax.experimental.pallas import tpu_sc as plsc

B = 4096
D = 64
LEN_VOCAB = 51
OUT_W = 3 * D + LEN_VOCAB
PAD_W = 256

_info = plsc.get_sparse_core_info()
NC = _info.num_cores
NS = _info.num_subcores
NW = NC * NS
BPW = B // NW


def _sc_adv_body(adv_id, adv_t, g_adv, ia, ra, sa):
    wid = lax.axis_index("s") * NC + lax.axis_index("c")
    base = wid * BPW
    pltpu.sync_copy(adv_id.at[pl.ds(base, BPW)], ia)
    pltpu.async_copy(adv_t.at[ia], ra, sa).wait()
    pltpu.sync_copy(ra, g_adv.at[pl.ds(base, BPW)])


def _sc_main_body(brd_id, ind_id, len_id, brd_t, ind_t, g_adv,
                  out, ib, ii, il, ra, rb, ri, out_v, sb, si):
    wid = lax.axis_index("s") * NC + lax.axis_index("c")
    base = wid * BPW

    pltpu.sync_copy(brd_id.at[pl.ds(base, BPW)], ib)
    pltpu.sync_copy(ind_id.at[pl.ds(base, BPW)], ii)
    pltpu.sync_copy(len_id.at[pl.ds(base, BPW)], il)

    cb = pltpu.async_copy(brd_t.at[ib], rb, sb)
    ci = pltpu.async_copy(ind_t.at[ii], ri, si)
    pltpu.sync_copy(g_adv.at[pl.ds(base, BPW)], ra)
    cb.wait()
    ci.wait()

    iota = lax.iota(jnp.int32, 16)
    one = jnp.full((16,), 1.0, jnp.float32)
    zero = jnp.zeros((16,), jnp.float32)

    def body(g, carry):
        s = g * 16
        vl = il[pl.ds(s, 16)]
        for j in range(16):
            r = s + j
            for c in range(4):
                out_v[r, pl.ds(c * 16, 16)] = ra[r, pl.ds(c * 16, 16)]
            for c in range(4):
                out_v[r, pl.ds(D + c * 16, 16)] = rb[r, pl.ds(c * 16, 16)]
            for c in range(4):
                out_v[r, pl.ds(2 * D + c * 16, 16)] = jnp.where(
                    iota + (c * 16) == vl[j], one, zero)
            for c in range(4):
                out_v[r, pl.ds(2 * D + LEN_VOCAB + c * 16, 16)] = \
                    ri[r, pl.ds(c * 16, 16)]
        return carry

    lax.fori_loop(0, BPW // 16, body, 0)
    pltpu.sync_copy(out_v, out.at[pl.ds(base, BPW)])


def kernel(advertiser_id, brand_id, industry, campaign_length,
           adv_table, brand_table, ind_table):
    mesh = plsc.VectorSubcoreMesh(core_axis_name="c", subcore_axis_name="s")
    adv_gather = pl.kernel(
        _sc_adv_body,
        mesh=mesh,
        compiler_params=pltpu.CompilerParams(use_tc_tiling_on_sc=False),
        out_type=jax.ShapeDtypeStruct((B, D), jnp.float32),
        scratch_types=[
            pltpu.VMEM((BPW,), jnp.int32),
            pltpu.VMEM((BPW, D), jnp.float32),
            pltpu.SemaphoreType.DMA,
        ],
    )
    g_adv = adv_gather(advertiser_id, adv_table)

    main = pl.kernel(
        _sc_main_body,
        mesh=mesh,
        compiler_params=pltpu.CompilerParams(use_tc_tiling_on_sc=False),
        out_type=jax.ShapeDtypeStruct((B, PAD_W), jnp.float32),
        scratch_types=[
            pltpu.VMEM((BPW,), jnp.int32),
            pltpu.VMEM((BPW,), jnp.int32),
            pltpu.VMEM((BPW,), jnp.int32),
            pltpu.VMEM((BPW, D), jnp.float32),
            pltpu.VMEM((BPW, D), jnp.float32),
            pltpu.VMEM((BPW, D), jnp.float32),
            pltpu.VMEM((BPW, PAD_W), jnp.float32),
            pltpu.SemaphoreType.DMA,
            pltpu.SemaphoreType.DMA,
        ],
    )
    padded = main(brand_id, industry, campaign_length,
                  brand_table, ind_table, g_adv)
    return padded[:, :OUT_W]
